# Initial kernel scaffold; baseline (speedup 1.0000x reference)
import jax
import jax.numpy as jnp
from jax.experimental import pallas as pl


def kernel(x, edge_index, W_in, b_in, W1l, b1l, W1r, g1, be1, W2l, b2l, W2r, g2, be2, W3l, b3l, W3r, g3, be3, Wf1, bf1, Wf2, bf2):
    raise NotImplementedError("write your pallas kernel here")


# speedup vs baseline: 5.1489x; 5.1489x over previous
import functools

import jax
import jax.numpy as jnp
from jax import lax
from jax.experimental import pallas as pl
from jax.experimental.pallas import tpu as pltpu
from jax.experimental.pallas import tpu_sc as plsc

N = 10000
E = 320000
D = 128
NW = 32
NPW = 320
NPAD = NW * NPW
K = 256

NCHUNK = E // K
NF = D // 16
SUP = 10
NSUPER = NCHUNK // SUP

_mesh = plsc.VectorSubcoreMesh(core_axis_name="c", subcore_axis_name="s")


@functools.partial(
    pl.kernel,
    out_type=jax.ShapeDtypeStruct((NPAD, D), jnp.float32),
    mesh=_mesh,
    scratch_types=[
        pltpu.VMEM((SUP * K,), jnp.int32),
        pltpu.VMEM((2 * K, D), jnp.float32),
        pltpu.VMEM((SUP * K,), jnp.int32),
        pltpu.VMEM((16,), jnp.int32),
        pltpu.VMEM((NPW + 8, D), jnp.float32),
        pltpu.VMEM((1, D), jnp.float32),
        pltpu.SemaphoreType.DMA,
        pltpu.SemaphoreType.DMA,
    ],
)
def _sc_segmax(h_hbm, src_hbm, ctrl_hbm, rs_hbm, out_hbm,
               idx_v, rows_v, ctrl_v, rsw_v, out_v, acc_v, sem0, sem1):
    wid = lax.axis_index("s") * 2 + lax.axis_index("c")
    n0 = wid * NPW
    pltpu.sync_copy(rs_hbm.at[pl.ds(wid * 8, 16)], rsw_v)
    rsw = rsw_v[pl.ds(0, 16)]
    e0 = rsw[0]
    e1 = rsw[8]
    c_lo = e0 // K

    neg_inf = jnp.full((16,), -jnp.inf, jnp.float32)
    zeros16 = jnp.zeros((16,), jnp.float32)
    sent2 = (n0 + NPW) * 2 + 1

    def zero_body(i, _):
        for f in range(NF):
            out_v[i, pl.ds(f * 16, 16)] = zeros16
        return 0
    lax.fori_loop(0, NPW + 8, zero_body, 0)

    for f in range(NF):
        acc_v[0, pl.ds(f * 16, 16)] = neg_inf

    def fire(kk, par):
        pltpu.async_copy(
            h_hbm.at[idx_v.at[pl.ds(kk * K, K)]],
            rows_v.at[pl.ds(par * K, K)],
            sem0 if par == 0 else sem1)

    def wait(par):
        pltpu.make_async_copy(
            h_hbm.at[idx_v.at[pl.ds(0, K)]],
            rows_v.at[pl.ds(par * K, K)],
            sem0 if par == 0 else sem1).wait()

    def process(base, koff, poff, _):
        acc = tuple(acc_v[0, pl.ds(f * 16, 16)] for f in range(NF))

        def group_body(g, acc):
            win = ctrl_v[pl.ds(koff + g * 16, 16)]
            ebase = base + g * 16
            for j in range(16):
                e = ebase + j
                active = jnp.logical_and(e >= e0, e < e1)
                d2 = jnp.where(active, win[j], sent2)
                row = poff + g * 16 + j
                acc = tuple(
                    jnp.maximum(acc[f], rows_v[row, pl.ds(f * 16, 16)])
                    for f in range(NF)
                )

                is_end = d2 & 1 == 1
                tgt = jnp.where(is_end, (d2 >> 1) - n0, NPW)
                for f in range(NF):
                    out_v[tgt, pl.ds(f * 16, 16)] = acc[f]
                acc = tuple(
                    jnp.where(is_end, neg_inf, acc[f]) for f in range(NF)
                )
            return acc

        acc = lax.fori_loop(0, K // 16, group_body, acc)
        for f in range(NF):
            acc_v[0, pl.ds(f * 16, 16)] = acc[f]
        return 0

    def pair_body(i, carry):
        c0 = 2 * i
        s = c0 // SUP
        k0 = c0 - s * SUP
        base0 = c0 * K
        base1 = base0 + K
        sup_base = s * SUP * K
        ov0 = jnp.logical_and(base0 < e1, base0 + K > e0)
        ov1 = jnp.logical_and(base1 < e1, base1 + K > e0)
        ov2 = jnp.logical_and(base1 + K < e1, base1 + 2 * K > e0)

        def load_super(_):
            pltpu.sync_copy(src_hbm.at[pl.ds(sup_base, SUP * K)], idx_v)
            pltpu.sync_copy(ctrl_hbm.at[pl.ds(sup_base, SUP * K)], ctrl_v)
            return 0

        sup_overlap = jnp.logical_and(sup_base < e1, sup_base + SUP * K > e0)
        lax.cond(jnp.logical_and(k0 == 0, sup_overlap), load_super,
                 lambda _: 0, 0)

        def fire0_self(_):
            fire(k0, 0)
            return 0
        lax.cond(jnp.logical_and(ov0, jnp.logical_or(k0 == 0, c0 == c_lo)),
                 fire0_self, lambda _: 0, 0)

        def blk0(_):
            def pre(_):
                fire(k0 + 1, 1)
                return 0
            lax.cond(ov1, pre, lambda _: 0, 0)
            wait(0)
            return process(base0, k0 * K, 0, _)
        lax.cond(ov0, blk0, lambda _: 0, 0)

        def fire1_self(_):
            fire(k0 + 1, 1)
            return 0
        lax.cond(jnp.logical_and(ov1, jnp.logical_not(ov0)), fire1_self,
                 lambda _: 0, 0)

        def blk1(_):
            def pre(_):
                fire(k0 + 2, 0)
                return 0
            lax.cond(jnp.logical_and(ov2, k0 + 2 < SUP), pre,
                     lambda _: 0, 0)
            wait(1)
            return process(base1, (k0 + 1) * K, K, _)
        lax.cond(ov1, blk1, lambda _: 0, 0)
        return carry

    lax.fori_loop(0, NCHUNK // 2, pair_body, 0)
    pltpu.sync_copy(out_v.at[pl.ds(0, NPW)], out_hbm.at[pl.ds(n0, NPW)])


def _tc_call(body, out_rows, out_cols, *args):
    return pl.pallas_call(
        body,
        out_shape=jax.ShapeDtypeStruct((out_rows, out_cols), jnp.float32),
    )(*args)


def _proj_body(x_ref, w_ref, b_ref, o_ref):
    o_ref[...] = (
        jnp.dot(x_ref[...], w_ref[...].T, preferred_element_type=jnp.float32)
        + b_ref[...]
    )


def _layer_body(h_ref, agg_ref, wl_ref, bl_ref, wr_ref, g_ref, be_ref, o_ref):
    z = (
        jnp.dot(agg_ref[...], wl_ref[...].T, preferred_element_type=jnp.float32)
        + bl_ref[...]
        + jnp.dot(h_ref[...], wr_ref[...].T, preferred_element_type=jnp.float32)
    )
    m = jnp.mean(z, axis=0, keepdims=True)
    v = jnp.mean((z - m) * (z - m), axis=0, keepdims=True)
    zn = (z - m) * lax.rsqrt(v + 1e-5) * g_ref[...] + be_ref[...]
    o_ref[...] = jnp.where(zn >= 0, zn, 0.01 * zn)


def _last_body(h_ref, agg_ref, wl_ref, bl_ref, wr_ref, g_ref, be_ref,
               res_ref, wf1_ref, bf1_ref, wf2_ref, bf2_ref, o_ref):
    z = (
        jnp.dot(agg_ref[...], wl_ref[...].T, preferred_element_type=jnp.float32)
        + bl_ref[...]
        + jnp.dot(h_ref[...], wr_ref[...].T, preferred_element_type=jnp.float32)
    )
    m = jnp.mean(z, axis=0, keepdims=True)
    v = jnp.mean((z - m) * (z - m), axis=0, keepdims=True)
    zn = (z - m) * lax.rsqrt(v + 1e-5) * g_ref[...] + be_ref[...]
    h = jnp.where(zn >= 0, zn, 0.01 * zn) + res_ref[...]
    t = jnp.dot(h, wf1_ref[...].T, preferred_element_type=jnp.float32) + bf1_ref[...]
    t = jnp.where(t >= 0, t, 0.01 * t)
    o_ref[...] = (
        jnp.dot(t, wf2_ref[...].T, preferred_element_type=jnp.float32) + bf2_ref[...]
    )


def kernel(x, edge_index, W_in, b_in, W1l, b1l, W1r, g1, be1, W2l, b2l, W2r,
           g2, be2, W3l, b3l, W3r, g3, be3, Wf1, bf1, Wf2, bf2):
    src, dst = edge_index[0], edge_index[1]
    order = jnp.argsort(dst)
    src_s = src[order]
    dst_s = jnp.sort(dst)
    dst_next = jnp.concatenate([dst_s[1:], jnp.full((1,), N, jnp.int32)])
    ctrl = dst_s * 2 + (dst_s != dst_next).astype(jnp.int32)
    node_bounds = jnp.minimum(jnp.arange(33, dtype=jnp.int32) * NPW, N)
    eb = jnp.searchsorted(dst_s, node_bounds).astype(jnp.int32)
    rs8 = jnp.zeros((34, 8), jnp.int32).at[:33, 0].set(eb).reshape(-1)

    r2 = lambda a: a.reshape(1, -1)

    h = _tc_call(_proj_body, N, D, x, W_in, r2(b_in))
    residual = h

    for Wl, bl, Wr, g, be in ((W1l, b1l, W1r, g1, be1),
                              (W2l, b2l, W2r, g2, be2)):
        agg = _sc_segmax(h, src_s, ctrl, rs8)[:N]
        h = _tc_call(_layer_body, N, D, h, agg, Wl, r2(bl), Wr, r2(g), r2(be))

    agg = _sc_segmax(h, src_s, ctrl, rs8)[:N]
    out = pl.pallas_call(
        _last_body,
        out_shape=jax.ShapeDtypeStruct((N, 64), jnp.float32),
    )(h, agg, W3l, r2(b3l), W3r, r2(g3), r2(be3),
      residual, Wf1, r2(bf1), Wf2, r2(bf2))
    return out


# speedup vs baseline: 5.2502x; 1.0197x over previous
import functools

import jax
import jax.numpy as jnp
from jax import lax
from jax.experimental import pallas as pl
from jax.experimental.pallas import tpu as pltpu
from jax.experimental.pallas import tpu_sc as plsc

N = 10000
E = 320000
D = 128
NW = 32
NPW = 320
NPAD = NW * NPW
K = 256

NCHUNK = E // K
NF = D // 16
SUP = 10
NSUPER = NCHUNK // SUP

_mesh = plsc.VectorSubcoreMesh(core_axis_name="c", subcore_axis_name="s")


@functools.partial(
    pl.kernel,
    out_type=jax.ShapeDtypeStruct((NPAD, D), jnp.float32),
    mesh=_mesh,
    scratch_types=[
        pltpu.VMEM((SUP * K,), jnp.int32),
        pltpu.VMEM((2 * K, D), jnp.float32),
        pltpu.VMEM((SUP * K,), jnp.int32),
        pltpu.VMEM((16,), jnp.int32),
        pltpu.VMEM((NPW + 8, D), jnp.float32),
        pltpu.VMEM((1, D), jnp.float32),
        pltpu.SemaphoreType.DMA,
        pltpu.SemaphoreType.DMA,
    ],
)
def _sc_segmax(h_hbm, src_hbm, ctrl_hbm, rs_hbm, out_hbm,
               idx_v, rows_v, ctrl_v, rsw_v, out_v, acc_v, sem0, sem1):
    wid = lax.axis_index("s") * 2 + lax.axis_index("c")
    n0 = wid * NPW
    pltpu.sync_copy(rs_hbm.at[pl.ds(wid * 8, 16)], rsw_v)
    rsw = rsw_v[pl.ds(0, 16)]
    e0 = rsw[0]
    e1 = rsw[8]
    c_lo = e0 // K

    neg_inf = jnp.full((16,), -jnp.inf, jnp.float32)
    zeros16 = jnp.zeros((16,), jnp.float32)
    sent2 = (n0 + NPW) * 2 + 1

    def zero_body(i, _):
        for f in range(NF):
            out_v[i, pl.ds(f * 16, 16)] = zeros16
        return 0
    lax.fori_loop(0, NPW + 8, zero_body, 0)

    for f in range(NF):
        acc_v[0, pl.ds(f * 16, 16)] = neg_inf

    def fire(kk, par):
        pltpu.async_copy(
            h_hbm.at[idx_v.at[pl.ds(kk * K, K)]],
            rows_v.at[pl.ds(par * K, K)],
            sem0 if par == 0 else sem1)

    def wait(par):
        pltpu.make_async_copy(
            h_hbm.at[idx_v.at[pl.ds(0, K)]],
            rows_v.at[pl.ds(par * K, K)],
            sem0 if par == 0 else sem1).wait()

    def process(base, koff, poff, _):
        acc = tuple(acc_v[0, pl.ds(f * 16, 16)] for f in range(NF))

        def group_body(g, acc):
            win = ctrl_v[pl.ds(koff + g * 16, 16)]
            ebase = base + g * 16
            for j in range(16):
                e = ebase + j
                active = jnp.logical_and(e >= e0, e < e1)
                d2 = jnp.where(active, win[j], sent2)
                row = poff + g * 16 + j
                acc = tuple(
                    jnp.maximum(acc[f], rows_v[row, pl.ds(f * 16, 16)])
                    for f in range(NF)
                )

                is_end = d2 & 1 == 1
                tgt = jnp.where(is_end, (d2 >> 1) - n0, NPW)
                for f in range(NF):
                    out_v[tgt, pl.ds(f * 16, 16)] = acc[f]
                acc = tuple(
                    jnp.where(is_end, neg_inf, acc[f]) for f in range(NF)
                )
            return acc

        acc = lax.fori_loop(0, K // 16, group_body, acc)
        for f in range(NF):
            acc_v[0, pl.ds(f * 16, 16)] = acc[f]
        return 0

    def pair_body(i, carry):
        c0 = 2 * i
        s = c0 // SUP
        k0 = c0 - s * SUP
        base0 = c0 * K
        base1 = base0 + K
        sup_base = s * SUP * K
        ov0 = jnp.logical_and(base0 < e1, base0 + K > e0)
        ov1 = jnp.logical_and(base1 < e1, base1 + K > e0)
        ov2 = jnp.logical_and(base1 + K < e1, base1 + 2 * K > e0)

        def load_super(_):
            pltpu.sync_copy(src_hbm.at[pl.ds(sup_base, SUP * K)], idx_v)
            pltpu.sync_copy(ctrl_hbm.at[pl.ds(sup_base, SUP * K)], ctrl_v)
            return 0

        sup_overlap = jnp.logical_and(sup_base < e1, sup_base + SUP * K > e0)
        lax.cond(jnp.logical_and(k0 == 0, sup_overlap), load_super,
                 lambda _: 0, 0)

        def fire0_self(_):
            fire(k0, 0)
            return 0
        lax.cond(jnp.logical_and(ov0, jnp.logical_or(k0 == 0, c0 == c_lo)),
                 fire0_self, lambda _: 0, 0)

        def blk0(_):
            def pre(_):
                fire(k0 + 1, 1)
                return 0
            lax.cond(ov1, pre, lambda _: 0, 0)
            wait(0)
            return process(base0, k0 * K, 0, _)
        lax.cond(ov0, blk0, lambda _: 0, 0)

        def fire1_self(_):
            fire(k0 + 1, 1)
            return 0
        lax.cond(jnp.logical_and(ov1, jnp.logical_not(ov0)), fire1_self,
                 lambda _: 0, 0)

        def blk1(_):
            def pre(_):
                fire(k0 + 2, 0)
                return 0
            lax.cond(jnp.logical_and(ov2, k0 + 2 < SUP), pre,
                     lambda _: 0, 0)
            wait(1)
            return process(base1, (k0 + 1) * K, K, _)
        lax.cond(ov1, blk1, lambda _: 0, 0)
        return carry

    lax.fori_loop(0, NCHUNK // 2, pair_body, 0)
    pltpu.sync_copy(out_v.at[pl.ds(0, NPW)], out_hbm.at[pl.ds(n0, NPW)])


def _tc_call(body, out_rows, out_cols, *args):
    return pl.pallas_call(
        body,
        out_shape=jax.ShapeDtypeStruct((out_rows, out_cols), jnp.float32),
    )(*args)


def _proj_body(x_ref, w_ref, b_ref, o_ref):
    o_ref[...] = (
        jnp.dot(x_ref[...], w_ref[...].T, preferred_element_type=jnp.float32)
        + b_ref[...]
    )


def _layer_body(h_ref, agg_ref, wl_ref, bl_ref, wr_ref, g_ref, be_ref, o_ref):
    z = (
        jnp.dot(agg_ref[...], wl_ref[...].T, preferred_element_type=jnp.float32)
        + bl_ref[...]
        + jnp.dot(h_ref[...], wr_ref[...].T, preferred_element_type=jnp.float32)
    )
    m = jnp.mean(z, axis=0, keepdims=True)
    v = jnp.mean((z - m) * (z - m), axis=0, keepdims=True)
    zn = (z - m) * lax.rsqrt(v + 1e-5) * g_ref[...] + be_ref[...]
    o_ref[...] = jnp.where(zn >= 0, zn, 0.01 * zn)


def _last_body(h_ref, agg_ref, wl_ref, bl_ref, wr_ref, g_ref, be_ref,
               res_ref, wf1_ref, bf1_ref, wf2_ref, bf2_ref, o_ref):
    z = (
        jnp.dot(agg_ref[...], wl_ref[...].T, preferred_element_type=jnp.float32)
        + bl_ref[...]
        + jnp.dot(h_ref[...], wr_ref[...].T, preferred_element_type=jnp.float32)
    )
    m = jnp.mean(z, axis=0, keepdims=True)
    v = jnp.mean((z - m) * (z - m), axis=0, keepdims=True)
    zn = (z - m) * lax.rsqrt(v + 1e-5) * g_ref[...] + be_ref[...]
    h = jnp.where(zn >= 0, zn, 0.01 * zn) + res_ref[...]
    t = jnp.dot(h, wf1_ref[...].T, preferred_element_type=jnp.float32) + bf1_ref[...]
    t = jnp.where(t >= 0, t, 0.01 * t)
    o_ref[...] = (
        jnp.dot(t, wf2_ref[...].T, preferred_element_type=jnp.float32) + bf2_ref[...]
    )


def kernel(x, edge_index, W_in, b_in, W1l, b1l, W1r, g1, be1, W2l, b2l, W2r,
           g2, be2, W3l, b3l, W3r, g3, be3, Wf1, bf1, Wf2, bf2):
    src, dst = edge_index[0], edge_index[1]
    packed = jnp.sort(dst * 16384 + src)
    src_s = packed & 16383
    dst_s = packed >> 14
    dst_next = jnp.concatenate([dst_s[1:], jnp.full((1,), N, jnp.int32)])
    ctrl = dst_s * 2 + (dst_s != dst_next).astype(jnp.int32)
    node_bounds = jnp.minimum(jnp.arange(33, dtype=jnp.int32) * NPW, N)
    eb = jnp.searchsorted(dst_s, node_bounds).astype(jnp.int32)
    rs8 = jnp.zeros((34, 8), jnp.int32).at[:33, 0].set(eb).reshape(-1)

    r2 = lambda a: a.reshape(1, -1)

    h = _tc_call(_proj_body, N, D, x, W_in, r2(b_in))
    residual = h

    for Wl, bl, Wr, g, be in ((W1l, b1l, W1r, g1, be1),
                              (W2l, b2l, W2r, g2, be2)):
        agg = _sc_segmax(h, src_s, ctrl, rs8)[:N]
        h = _tc_call(_layer_body, N, D, h, agg, Wl, r2(bl), Wr, r2(g), r2(be))

    agg = _sc_segmax(h, src_s, ctrl, rs8)[:N]
    out = pl.pallas_call(
        _last_body,
        out_shape=jax.ShapeDtypeStruct((N, 64), jnp.float32),
    )(h, agg, W3l, r2(b3l), W3r, r2(g3), r2(be3),
      residual, Wf1, r2(bf1), Wf2, r2(bf2))
    return out
